# baseline (speedup 1.0000x reference)
import functools

import jax
import jax.numpy as jnp
from jax import lax
from jax.experimental import pallas as pl
from jax.experimental.pallas import tpu as pltpu
from jax.experimental.pallas import tpu_sc as plsc

_NUM_EMB = 1_000_000
_D = 64
_B = 4096
_L = 200
_N = _B * _L
_NW = 32
_PER_W = _N // _NW
_C = 128
_NCHUNK = _PER_W // _C


def _embed_gather(idx_flat, weight):
    mesh = plsc.VectorSubcoreMesh(core_axis_name="c", subcore_axis_name="s")

    @functools.partial(
        pl.kernel,
        mesh=mesh,
        out_type=jax.ShapeDtypeStruct((_N, _D), jnp.float32),
        compiler_params=pltpu.CompilerParams(use_tc_tiling_on_sc=False),
        scratch_types=[
            pltpu.VMEM((_C,), jnp.int32),
            pltpu.VMEM((_C, _D), jnp.float32),
            pltpu.SemaphoreType.DMA,
        ],
    )
    def k(idx_hbm, table_hbm, out_hbm, idx_v, rows_v, gsem):
        wid = lax.axis_index("s") * 2 + lax.axis_index("c")
        base = wid * _PER_W

        def step(i, carry):
            off = base + i * _C
            pltpu.sync_copy(idx_hbm.at[pl.ds(off, _C)], idx_v)
            pltpu.async_copy(table_hbm.at[idx_v], rows_v, gsem).wait()
            pltpu.sync_copy(rows_v, out_hbm.at[pl.ds(off, _C)])
            return carry

        lax.fori_loop(0, _NCHUNK, step, 0, unroll=False)

    return k(idx_flat, weight)


def kernel(input, weight):
    idx_flat = input.reshape(_N)
    out = _embed_gather(idx_flat, weight)
    return out.reshape(_B, _L, _D)


# speedup vs baseline: 1.2197x; 1.2197x over previous
import functools

import jax
import jax.numpy as jnp
from jax import lax
from jax.experimental import pallas as pl
from jax.experimental.pallas import tpu as pltpu
from jax.experimental.pallas import tpu_sc as plsc

_NUM_EMB = 1_000_000
_D = 64
_B = 4096
_L = 200
_N = _B * _L
_NW = 32
_PER_W = _N // _NW
_C = 128
_NCHUNK = _PER_W // _C
_S = 8
_NGROUP = _NCHUNK // _S


def _embed_gather(idx_flat, weight):
    mesh = plsc.VectorSubcoreMesh(core_axis_name="c", subcore_axis_name="s")

    @functools.partial(
        pl.kernel,
        mesh=mesh,
        out_type=jax.ShapeDtypeStruct((_N, _D), jnp.float32),
        compiler_params=pltpu.CompilerParams(use_tc_tiling_on_sc=False),
        scratch_types=(
            [pltpu.VMEM((_PER_W,), jnp.int32)]
            + [pltpu.VMEM((_C, _D), jnp.float32) for _ in range(_S)]
            + [pltpu.SemaphoreType.DMA for _ in range(2 * _S)]
        ),
    )
    def k(idx_hbm, table_hbm, out_hbm, idx_v, *bufs_and_sems):
        rows = bufs_and_sems[:_S]
        gsem = bufs_and_sems[_S:2 * _S]
        osem = bufs_and_sems[2 * _S:]
        wid = lax.axis_index("s") * 2 + lax.axis_index("c")
        base = wid * _PER_W

        pltpu.sync_copy(idx_hbm.at[pl.ds(base, _PER_W)], idx_v)

        for b in range(_S):
            pltpu.async_copy(
                table_hbm.at[idx_v.at[pl.ds(b * _C, _C)]], rows[b], gsem[b])

        def group(g, carry):
            out_copies = []
            for b in range(_S):
                i = g * _S + b
                pltpu.make_async_copy(
                    table_hbm.at[idx_v.at[pl.ds(0, _C)]], rows[b],
                    gsem[b]).wait()
                out_copies.append(pltpu.async_copy(
                    rows[b], out_hbm.at[pl.ds(base + i * _C, _C)], osem[b]))
            for b in range(_S):
                @pl.when(g + 1 < _NGROUP)
                def _():
                    i2 = (g + 1) * _S + b
                    out_copies[b].wait()
                    pltpu.async_copy(
                        table_hbm.at[idx_v.at[pl.ds(i2 * _C, _C)]], rows[b],
                        gsem[b])
            return carry

        lax.fori_loop(0, _NGROUP, group, 0, unroll=False)

        for b in range(_S):
            pltpu.make_async_copy(
                rows[b], out_hbm.at[pl.ds(base, _C)], osem[b]).wait()

    return k(idx_flat, weight)


def kernel(input, weight):
    idx_flat = input.T.reshape(_N)
    out = _embed_gather(idx_flat, weight)
    return out.reshape(_L, _B, _D).transpose(1, 0, 2)
